# baseline (device time: 11801 ns/iter reference)
import jax
import jax.numpy as jnp
from jax import lax
from jax.experimental import pallas as pl
from jax.experimental.pallas import tpu as pltpu

N_DEV = 4
VC = 1024


def kernel(x, W, labels):
    T, D = x.shape
    _, V = W.shape
    K = V // VC

    def body(x_hbm, w_hbm, lab_hbm, out_hbm,
             x_ref, w_bufs, lab_ref, out_vmem, comm_ref,
             in_sems, w_sems, out_sem, send_sems, recv_sems):
        my_pos = lax.axis_index("i")
        barrier_sem = pltpu.get_barrier_semaphore()
        for j in range(1, N_DEV):
            peer = lax.rem(my_pos + j, N_DEV)
            pl.semaphore_signal(
                barrier_sem, inc=1,
                device_id=(peer,), device_id_type=pl.DeviceIdType.MESH,
            )

        cx = pltpu.make_async_copy(x_hbm, x_ref, in_sems.at[0])
        cl = pltpu.make_async_copy(lab_hbm, lab_ref, in_sems.at[1])
        cx.start()
        cl.start()
        w_copies = [
            pltpu.make_async_copy(
                w_hbm.at[:, pl.ds(k * VC, VC)], w_bufs.at[k % 2],
                w_sems.at[k % 2],
            )
            for k in range(K)
        ]
        w_copies[0].start()
        w_copies[1].start()

        cx.wait()
        xb = x_ref[...].astype(jnp.bfloat16)
        cl.wait()
        lab_row = lab_ref[...].reshape(1, T)

        acc = None
        for k in range(K):
            w_copies[k].wait()
            wb = w_bufs[k % 2].astype(jnp.bfloat16)
            logits_t = lax.dot_general(
                wb, xb,
                dimension_numbers=(((0,), (1,)), ((), ())),
                preferred_element_type=jnp.float32,
            ).astype(jnp.bfloat16)

            e_t = jnp.exp(logits_t)
            vio = lax.broadcasted_iota(jnp.int32, (VC, T), 0) + (
                my_pos * V + k * VC
            )
            masked_t = jnp.where(
                vio == lab_row, logits_t, jnp.bfloat16(0.0)
            )
            ones8 = jnp.ones((8, VC), jnp.bfloat16)
            s8 = lax.dot_general(
                ones8, e_t, dimension_numbers=(((1,), (0,)), ((), ())),
                preferred_element_type=jnp.float32,
            )
            c8 = lax.dot_general(
                ones8, masked_t,
                dimension_numbers=(((1,), (0,)), ((), ())),
                preferred_element_type=jnp.float32,
            )
            part = jnp.concatenate([s8[0:1], c8[0:1]], axis=0)
            acc = part if acc is None else acc + part
            if k + 2 < K:
                w_copies[k + 2].start()

        comm_ref[0] = acc
        pl.semaphore_wait(barrier_sem, N_DEV - 1)

        rdmas = []
        for j in range(1, N_DEV):
            peer = lax.rem(my_pos + j, N_DEV)
            rdma = pltpu.make_async_remote_copy(
                src_ref=comm_ref.at[0],
                dst_ref=comm_ref.at[j],
                send_sem=send_sems.at[j - 1],
                recv_sem=recv_sems.at[j - 1],
                device_id=(peer,),
                device_id_type=pl.DeviceIdType.MESH,
            )
            rdma.start()
            rdmas.append(rdma)
        for rdma in rdmas:
            rdma.wait_recv()

        tot = comm_ref[0] + comm_ref[1] + comm_ref[2] + comm_ref[3]
        out_vmem[...] = (jnp.log(tot[0:1]) - tot[1:2]).reshape(T)
        co = pltpu.make_async_copy(out_vmem, out_hbm, out_sem)
        co.start()
        co.wait()

        for rdma in rdmas:
            rdma.wait_send()

    out = pl.pallas_call(
        body,
        out_shape=jax.ShapeDtypeStruct((T,), jnp.float32),
        in_specs=[
            pl.BlockSpec(memory_space=pltpu.MemorySpace.HBM),
            pl.BlockSpec(memory_space=pltpu.MemorySpace.HBM),
            pl.BlockSpec(memory_space=pltpu.MemorySpace.HBM),
        ],
        out_specs=pl.BlockSpec(memory_space=pltpu.MemorySpace.HBM),
        scratch_shapes=[
            pltpu.VMEM((T, D), jnp.float32),
            pltpu.VMEM((2, D, VC), jnp.float32),
            pltpu.VMEM((T,), jnp.int32),
            pltpu.VMEM((T,), jnp.float32),
            pltpu.VMEM((N_DEV, 2, T), jnp.float32),
            pltpu.SemaphoreType.DMA((2,)),
            pltpu.SemaphoreType.DMA((2,)),
            pltpu.SemaphoreType.DMA,
            pltpu.SemaphoreType.DMA((N_DEV - 1,)),
            pltpu.SemaphoreType.DMA((N_DEV - 1,)),
        ],
        compiler_params=pltpu.CompilerParams(collective_id=0),
    )(
        pltpu.with_memory_space_constraint(x, pltpu.MemorySpace.HBM),
        pltpu.with_memory_space_constraint(W, pltpu.MemorySpace.HBM),
        pltpu.with_memory_space_constraint(labels, pltpu.MemorySpace.HBM),
    )
    return out
